# baseline (speedup 1.0000x reference)
import functools

import jax
import jax.numpy as jnp
from jax import lax
from jax.experimental import pallas as pl
from jax.experimental.pallas import tpu as pltpu
from jax.experimental.pallas import tpu_sc as plsc

_NC = 2
_NS = 16
_NW = _NC * _NS
_CHUNK = 128


def _in2f_body(x_ref, w_ref, y_ref, yb_ref):
    y = jnp.dot(x_ref[...], w_ref[...], preferred_element_type=jnp.float32)
    y_ref[...] = y
    yb_ref[...] = y.astype(jnp.bfloat16)


def _assist_body(y_ref, j_ref, k_ref, rij_ref, rik_ref, m_ref, d_ref,
                 wft_ref, bft_ref, wfo_ref, bfo_ref, o_ref):
    _, ablk, nbh, g = d_ref.shape
    na, f = y_ref.shape[1], y_ref.shape[2]
    rij = rij_ref[0]
    rik = rik_ref[0]
    m = m_ref[0]
    inv = m / (rij + rik)
    cj = (rij * inv)[..., None].astype(jnp.bfloat16)
    ck = (rik * inv)[..., None].astype(jnp.bfloat16)
    jj = j_ref[0][..., None].astype(jnp.int16)
    kk = k_ref[0][..., None].astype(jnp.int16)
    aid = lax.broadcasted_iota(jnp.int16, (1, 1, na), 2)
    zero = jnp.zeros((), jnp.bfloat16)
    c_mat = (jnp.where(jj == aid, cj, zero)
             + jnp.where(kk == aid, ck, zero))
    z = jnp.dot(c_mat.reshape(ablk * nbh, na), y_ref[0],
                preferred_element_type=jnp.float32)
    wt = jnp.dot(d_ref[0].reshape(ablk * nbh, g), wft_ref[...],
                 preferred_element_type=jnp.float32)
    wt = wt + bft_ref[0][None, :]
    y2 = jnp.sum((wt * z).reshape(ablk, nbh, f), axis=1)
    o_ref[0] = (jnp.dot(y2, wfo_ref[...],
                        preferred_element_type=jnp.float32)
                + bfo_ref[0][None, :])


_NBUF = 5


def _make_gather(n_rows, f):
    n_chunks = n_rows // _CHUNK
    cpw = n_chunks // _NW
    n_iter = cpw // _NBUF
    mesh = plsc.VectorSubcoreMesh(core_axis_name="c", subcore_axis_name="s")

    @functools.partial(
        pl.kernel,
        mesh=mesh,
        out_type=jax.ShapeDtypeStruct((n_rows, f), jnp.float32),
        scratch_types=[
            pltpu.VMEM((cpw, 1, _CHUNK), jnp.int32),
            pltpu.VMEM((_NBUF, _CHUNK, f), jnp.float32),
        ] + [pltpu.SemaphoreType.DMA] * (2 * _NBUF),
    )
    def gather_kernel(table_hbm, idx_hbm, out_hbm, idx_v, rows_v, *sems):
        gsem = sems[:_NBUF]
        osem = sems[_NBUF:]
        wid = lax.axis_index("s") * _NC + lax.axis_index("c")
        base = wid * cpw
        pltpu.sync_copy(idx_hbm.at[pl.ds(base, cpw)], idx_v)

        def start_gather(bx, c):
            pltpu.async_copy(table_hbm.at[idx_v.at[c, 0]], rows_v.at[bx],
                             gsem[bx])

        def wait_gather(bx, c):
            pltpu.make_async_copy(table_hbm.at[idx_v.at[c, 0]], rows_v.at[bx],
                                  gsem[bx]).wait()

        def start_out(bx, c):
            row0 = (base + c) * _CHUNK
            pltpu.async_copy(rows_v.at[bx], out_hbm.at[pl.ds(row0, _CHUNK)],
                             osem[bx])

        def wait_out(bx):
            pltpu.make_async_copy(rows_v.at[bx],
                                  out_hbm.at[pl.ds(0, _CHUNK)],
                                  osem[bx]).wait()

        for bx in range(_NBUF):
            start_gather(bx, bx)

        def body(g, carry):
            c0 = g * _NBUF
            for bx in range(_NBUF):
                wait_gather(bx, c0 + bx)
                start_out(bx, c0 + bx)
            nxt = c0 + _NBUF

            @pl.when(nxt < cpw)
            def _refill():
                for bx in range(_NBUF):
                    wait_out(bx)
                    start_gather(bx, nxt + bx)

            return carry

        lax.fori_loop(0, n_iter, body, None)
        for bx in range(_NBUF):
            wait_out(bx)

    return gather_kernel


def _combine_body(yj_ref, yk_ref, rij_ref, rik_ref, m_ref, d_ref,
                  wft_ref, bft_ref, wfo_ref, bfo_ref, o_ref):
    _, ablk, nbh, g = d_ref.shape
    f = yj_ref.shape[4]
    rij = rij_ref[0]
    rik = rik_ref[0]
    m = m_ref[0]
    inv = m / (rij + rik)
    cj = (rij * inv)[:, None, :]
    ck = (rik * inv)[:, None, :]
    wt = jnp.dot(d_ref[0].reshape(ablk * nbh, g), wft_ref[...],
                 preferred_element_type=jnp.float32)
    wt = wt.reshape(ablk, nbh, f) + bft_ref[0][None, None, :]
    pj = wt * yj_ref[0, 0]
    pk = wt * yk_ref[0, 0]
    dn = (((2,), (1,)), ((0,), (0,)))
    y2 = (lax.dot_general(cj, pj, dn, preferred_element_type=jnp.float32)
          + lax.dot_general(ck, pk, dn,
                            preferred_element_type=jnp.float32))[:, 0, :]
    o_ref[0] = (jnp.dot(y2, wfo_ref[...],
                        preferred_element_type=jnp.float32)
                + bfo_ref[0][None, :])


def kernel(x, r_double, r_ij, r_ik, r_jk, neighbors, neighbor_mask,
           neighbors_j, neighbors_k, triple_masks, d_ijk, W_in2f, W_f2out,
           b_f2out, W_fd, b_fd, W_ft, b_ft):
    b, na, nbh = neighbors_j.shape
    f = x.shape[2]
    g = d_ijk.shape[3]

    y, y_b16 = pl.pallas_call(
        _in2f_body,
        out_shape=[jax.ShapeDtypeStruct((b * na, f), jnp.float32),
                   jax.ShapeDtypeStruct((b * na, f), jnp.bfloat16)],
    )(x.reshape(b * na, f), W_in2f)
    y_b16 = y_b16.reshape(b, na, f)

    na_sc = 640
    na_tc = na - na_sc
    n_parts = 4
    nap = na_sc // n_parts
    ablk = 40
    npb = nap // ablk
    rows_per_part = 2 * b * nap * nbh
    gather_fn = _make_gather(rows_per_part, f)
    boff = (jnp.arange(b, dtype=jnp.int32) * na)[:, None, None]

    outs = []
    for p in range(n_parts):
        a0 = p * nap
        idx_p = jnp.concatenate([
            (lax.dynamic_slice_in_dim(neighbors_j, a0, nap, 1) + boff
             ).reshape(-1),
            (lax.dynamic_slice_in_dim(neighbors_k, a0, nap, 1) + boff
             ).reshape(-1),
        ]).reshape(rows_per_part // _CHUNK, 1, _CHUNK)
        yjk_p = gather_fn(y, idx_p).reshape(2, b, nap, nbh, f)
        out_p = pl.pallas_call(
            _combine_body,
            grid=(b, npb),
            in_specs=[
                pl.BlockSpec((1, 1, ablk, nbh, f),
                             lambda bi, i: (0, bi, i, 0, 0)),
                pl.BlockSpec((1, 1, ablk, nbh, f),
                             lambda bi, i: (1, bi, i, 0, 0)),
                pl.BlockSpec((1, ablk, nbh),
                             lambda bi, i, p=p: (bi, p * npb + i, 0)),
                pl.BlockSpec((1, ablk, nbh),
                             lambda bi, i, p=p: (bi, p * npb + i, 0)),
                pl.BlockSpec((1, ablk, nbh),
                             lambda bi, i, p=p: (bi, p * npb + i, 0)),
                pl.BlockSpec((1, ablk, nbh, g),
                             lambda bi, i, p=p: (bi, p * npb + i, 0, 0)),
                pl.BlockSpec((g, f), lambda bi, i: (0, 0)),
                pl.BlockSpec((1, f), lambda bi, i: (0, 0)),
                pl.BlockSpec((f, f), lambda bi, i: (0, 0)),
                pl.BlockSpec((1, f), lambda bi, i: (0, 0)),
            ],
            out_specs=pl.BlockSpec((1, ablk, f), lambda bi, i: (bi, i, 0)),
            out_shape=jax.ShapeDtypeStruct((b, nap, f), jnp.float32),
            compiler_params=pltpu.CompilerParams(
                dimension_semantics=("parallel", "parallel")),
        )(yjk_p, yjk_p, r_ij, r_ik, triple_masks, d_ijk, W_ft,
          b_ft.reshape(1, f), W_f2out, b_f2out.reshape(1, f))
        outs.append(out_p)

    ablk2 = 8
    nb2 = na_tc // ablk2
    blk0 = na_sc // ablk2
    out_tc = pl.pallas_call(
        _assist_body,
        grid=(b, nb2),
        in_specs=[
            pl.BlockSpec((1, na, f), lambda bi, i: (bi, 0, 0)),
            pl.BlockSpec((1, ablk2, nbh), lambda bi, i: (bi, blk0 + i, 0)),
            pl.BlockSpec((1, ablk2, nbh), lambda bi, i: (bi, blk0 + i, 0)),
            pl.BlockSpec((1, ablk2, nbh), lambda bi, i: (bi, blk0 + i, 0)),
            pl.BlockSpec((1, ablk2, nbh), lambda bi, i: (bi, blk0 + i, 0)),
            pl.BlockSpec((1, ablk2, nbh), lambda bi, i: (bi, blk0 + i, 0)),
            pl.BlockSpec((1, ablk2, nbh, g),
                         lambda bi, i: (bi, blk0 + i, 0, 0)),
            pl.BlockSpec((g, f), lambda bi, i: (0, 0)),
            pl.BlockSpec((1, f), lambda bi, i: (0, 0)),
            pl.BlockSpec((f, f), lambda bi, i: (0, 0)),
            pl.BlockSpec((1, f), lambda bi, i: (0, 0)),
        ],
        out_specs=pl.BlockSpec((1, ablk2, f), lambda bi, i: (bi, i, 0)),
        out_shape=jax.ShapeDtypeStruct((b, na_tc, f), jnp.float32),
        compiler_params=pltpu.CompilerParams(
            dimension_semantics=("parallel", "parallel")),
    )(y_b16, neighbors_j, neighbors_k, r_ij, r_ik, triple_masks, d_ijk,
      W_ft, b_ft.reshape(1, f), W_f2out, b_f2out.reshape(1, f))
    outs.append(out_tc)
    return jnp.concatenate(outs, axis=1)


# speedup vs baseline: 1.0036x; 1.0036x over previous
import functools

import jax
import jax.numpy as jnp
from jax import lax
from jax.experimental import pallas as pl
from jax.experimental.pallas import tpu as pltpu
from jax.experimental.pallas import tpu_sc as plsc

_NC = 2
_NS = 16
_NW = _NC * _NS
_CHUNK = 128


def _in2f_body(x_ref, w_ref, y_ref, yb_ref):
    y = jnp.dot(x_ref[...], w_ref[...], preferred_element_type=jnp.float32)
    y_ref[...] = y
    yb_ref[...] = y.astype(jnp.bfloat16)


def _assist_body(y_ref, j_ref, k_ref, rij_ref, rik_ref, m_ref, d_ref,
                 wft_ref, bft_ref, wfo_ref, bfo_ref, o_ref):
    _, ablk, nbh, g = d_ref.shape
    na, f = y_ref.shape[1], y_ref.shape[2]
    rij = rij_ref[0]
    rik = rik_ref[0]
    m = m_ref[0]
    inv = m / (rij + rik)
    cj = (rij * inv)[..., None].astype(jnp.bfloat16)
    ck = (rik * inv)[..., None].astype(jnp.bfloat16)
    jj = j_ref[0][..., None].astype(jnp.int16)
    kk = k_ref[0][..., None].astype(jnp.int16)
    aid = lax.broadcasted_iota(jnp.int16, (1, 1, na), 2)
    zero = jnp.zeros((), jnp.bfloat16)
    c_mat = (jnp.where(jj == aid, cj, zero)
             + jnp.where(kk == aid, ck, zero))
    z = jnp.dot(c_mat.reshape(ablk * nbh, na), y_ref[0],
                preferred_element_type=jnp.float32)
    wt = jnp.dot(d_ref[0].reshape(ablk * nbh, g), wft_ref[...],
                 preferred_element_type=jnp.float32)
    wt = wt + bft_ref[0][None, :]
    y2 = jnp.sum((wt * z).reshape(ablk, nbh, f), axis=1)
    o_ref[0] = (jnp.dot(y2, wfo_ref[...],
                        preferred_element_type=jnp.float32)
                + bfo_ref[0][None, :])


_NBUF = 5


def _make_gather(n_rows, f):
    n_chunks = n_rows // _CHUNK
    cpw = n_chunks // _NW
    n_iter = cpw // _NBUF
    mesh = plsc.VectorSubcoreMesh(core_axis_name="c", subcore_axis_name="s")

    @functools.partial(
        pl.kernel,
        mesh=mesh,
        out_type=jax.ShapeDtypeStruct((n_rows, f), jnp.float32),
        scratch_types=[
            pltpu.VMEM((cpw, 1, _CHUNK), jnp.int32),
            pltpu.VMEM((_NBUF, _CHUNK, f), jnp.float32),
        ] + [pltpu.SemaphoreType.DMA] * (2 * _NBUF),
    )
    def gather_kernel(table_hbm, idx_hbm, out_hbm, idx_v, rows_v, *sems):
        gsem = sems[:_NBUF]
        osem = sems[_NBUF:]
        wid = lax.axis_index("s") * _NC + lax.axis_index("c")
        base = wid * cpw
        pltpu.sync_copy(idx_hbm.at[pl.ds(base, cpw)], idx_v)

        def start_gather(bx, c):
            pltpu.async_copy(table_hbm.at[idx_v.at[c, 0]], rows_v.at[bx],
                             gsem[bx])

        def wait_gather(bx, c):
            pltpu.make_async_copy(table_hbm.at[idx_v.at[c, 0]], rows_v.at[bx],
                                  gsem[bx]).wait()

        def start_out(bx, c):
            row0 = (base + c) * _CHUNK
            pltpu.async_copy(rows_v.at[bx], out_hbm.at[pl.ds(row0, _CHUNK)],
                             osem[bx])

        def wait_out(bx):
            pltpu.make_async_copy(rows_v.at[bx],
                                  out_hbm.at[pl.ds(0, _CHUNK)],
                                  osem[bx]).wait()

        for bx in range(_NBUF):
            start_gather(bx, bx)

        def body(g, carry):
            c0 = g * _NBUF
            for bx in range(_NBUF):
                wait_gather(bx, c0 + bx)
                start_out(bx, c0 + bx)
            nxt = c0 + _NBUF

            @pl.when(nxt < cpw)
            def _refill():
                for bx in range(_NBUF):
                    wait_out(bx)
                    start_gather(bx, nxt + bx)

            return carry

        lax.fori_loop(0, n_iter, body, None)
        for bx in range(_NBUF):
            wait_out(bx)

    return gather_kernel


def _combine_body(yj_ref, yk_ref, rij_ref, rik_ref, m_ref, d_ref,
                  wft_ref, bft_ref, wfo_ref, bfo_ref, o_ref):
    _, ablk, nbh, g = d_ref.shape
    f = yj_ref.shape[4]
    rij = rij_ref[0]
    rik = rik_ref[0]
    m = m_ref[0]
    inv = m / (rij + rik)
    cj = (rij * inv)[:, None, :]
    ck = (rik * inv)[:, None, :]
    wt = jnp.dot(d_ref[0].reshape(ablk * nbh, g), wft_ref[...],
                 preferred_element_type=jnp.float32)
    wt = wt.reshape(ablk, nbh, f) + bft_ref[0][None, None, :]
    pj = wt * yj_ref[0, 0]
    pk = wt * yk_ref[0, 0]
    dn = (((2,), (1,)), ((0,), (0,)))
    y2 = (lax.dot_general(cj, pj, dn, preferred_element_type=jnp.float32)
          + lax.dot_general(ck, pk, dn,
                            preferred_element_type=jnp.float32))[:, 0, :]
    o_ref[0] = (jnp.dot(y2, wfo_ref[...],
                        preferred_element_type=jnp.float32)
                + bfo_ref[0][None, :])


def kernel(x, r_double, r_ij, r_ik, r_jk, neighbors, neighbor_mask,
           neighbors_j, neighbors_k, triple_masks, d_ijk, W_in2f, W_f2out,
           b_f2out, W_fd, b_fd, W_ft, b_ft):
    b, na, nbh = neighbors_j.shape
    f = x.shape[2]
    g = d_ijk.shape[3]

    y, y_b16 = pl.pallas_call(
        _in2f_body,
        out_shape=[jax.ShapeDtypeStruct((b * na, f), jnp.float32),
                   jax.ShapeDtypeStruct((b * na, f), jnp.bfloat16)],
    )(x.reshape(b * na, f), W_in2f)
    y_b16 = y_b16.reshape(b, na, f)

    na_sc = 640
    na_tc = na - na_sc
    ablk = 40
    npb = na_sc // ablk
    rows_sc = 2 * b * na_sc * nbh
    boff = (jnp.arange(b, dtype=jnp.int32) * na)[:, None, None]

    outs = []
    idx_sc = jnp.concatenate([
        (lax.dynamic_slice_in_dim(neighbors_j, 0, na_sc, 1) + boff
         ).reshape(-1),
        (lax.dynamic_slice_in_dim(neighbors_k, 0, na_sc, 1) + boff
         ).reshape(-1),
    ]).reshape(rows_sc // _CHUNK, 1, _CHUNK)
    yjk = _make_gather(rows_sc, f)(y, idx_sc).reshape(2, b, na_sc, nbh, f)
    out_sc = pl.pallas_call(
        _combine_body,
        grid=(b, npb),
        in_specs=[
            pl.BlockSpec((1, 1, ablk, nbh, f),
                         lambda bi, i: (0, bi, i, 0, 0)),
            pl.BlockSpec((1, 1, ablk, nbh, f),
                         lambda bi, i: (1, bi, i, 0, 0)),
            pl.BlockSpec((1, ablk, nbh), lambda bi, i: (bi, i, 0)),
            pl.BlockSpec((1, ablk, nbh), lambda bi, i: (bi, i, 0)),
            pl.BlockSpec((1, ablk, nbh), lambda bi, i: (bi, i, 0)),
            pl.BlockSpec((1, ablk, nbh, g), lambda bi, i: (bi, i, 0, 0)),
            pl.BlockSpec((g, f), lambda bi, i: (0, 0)),
            pl.BlockSpec((1, f), lambda bi, i: (0, 0)),
            pl.BlockSpec((f, f), lambda bi, i: (0, 0)),
            pl.BlockSpec((1, f), lambda bi, i: (0, 0)),
        ],
        out_specs=pl.BlockSpec((1, ablk, f), lambda bi, i: (bi, i, 0)),
        out_shape=jax.ShapeDtypeStruct((b, na_sc, f), jnp.float32),
        compiler_params=pltpu.CompilerParams(
            dimension_semantics=("parallel", "parallel")),
    )(yjk, yjk, r_ij, r_ik, triple_masks, d_ijk, W_ft,
      b_ft.reshape(1, f), W_f2out, b_f2out.reshape(1, f))
    outs.append(out_sc)

    ablk2 = 8
    nb2 = na_tc // ablk2
    blk0 = na_sc // ablk2
    out_tc = pl.pallas_call(
        _assist_body,
        grid=(b, nb2),
        in_specs=[
            pl.BlockSpec((1, na, f), lambda bi, i: (bi, 0, 0)),
            pl.BlockSpec((1, ablk2, nbh), lambda bi, i: (bi, blk0 + i, 0)),
            pl.BlockSpec((1, ablk2, nbh), lambda bi, i: (bi, blk0 + i, 0)),
            pl.BlockSpec((1, ablk2, nbh), lambda bi, i: (bi, blk0 + i, 0)),
            pl.BlockSpec((1, ablk2, nbh), lambda bi, i: (bi, blk0 + i, 0)),
            pl.BlockSpec((1, ablk2, nbh), lambda bi, i: (bi, blk0 + i, 0)),
            pl.BlockSpec((1, ablk2, nbh, g),
                         lambda bi, i: (bi, blk0 + i, 0, 0)),
            pl.BlockSpec((g, f), lambda bi, i: (0, 0)),
            pl.BlockSpec((1, f), lambda bi, i: (0, 0)),
            pl.BlockSpec((f, f), lambda bi, i: (0, 0)),
            pl.BlockSpec((1, f), lambda bi, i: (0, 0)),
        ],
        out_specs=pl.BlockSpec((1, ablk2, f), lambda bi, i: (bi, i, 0)),
        out_shape=jax.ShapeDtypeStruct((b, na_tc, f), jnp.float32),
        compiler_params=pltpu.CompilerParams(
            dimension_semantics=("parallel", "parallel")),
    )(y_b16, neighbors_j, neighbors_k, r_ij, r_ik, triple_masks, d_ijk,
      W_ft, b_ft.reshape(1, f), W_f2out, b_f2out.reshape(1, f))
    outs.append(out_tc)
    return jnp.concatenate(outs, axis=1)


# speedup vs baseline: 1.0203x; 1.0166x over previous
import functools

import jax
import jax.numpy as jnp
from jax import lax
from jax.experimental import pallas as pl
from jax.experimental.pallas import tpu as pltpu
from jax.experimental.pallas import tpu_sc as plsc

_NC = 2
_NS = 16
_NW = _NC * _NS
_CHUNK = 128


def _in2f_body(x_ref, w_ref, j_ref, k_ref, y_ref, yb_ref, idx_ref):
    y = jnp.dot(x_ref[...], w_ref[...], preferred_element_type=jnp.float32)
    y_ref[...] = y
    yb_ref[...] = y.astype(jnp.bfloat16)
    b, na_sc, nbh = j_ref.shape
    na = y.shape[0] // b
    boff = lax.broadcasted_iota(jnp.int32, (b, na_sc, nbh), 0) * na
    half = b * na_sc
    idx_ref[:half, 0, :] = (j_ref[...] + boff).reshape(half, nbh)
    idx_ref[half:, 0, :] = (k_ref[...] + boff).reshape(half, nbh)


def _assist_body(y_ref, j_ref, k_ref, rij_ref, rik_ref, m_ref, d_ref,
                 wft_ref, bft_ref, wfo_ref, bfo_ref, o_ref):
    _, ablk, nbh, g = d_ref.shape
    na, f = y_ref.shape[1], y_ref.shape[2]
    rij = rij_ref[0]
    rik = rik_ref[0]
    m = m_ref[0]
    inv = m / (rij + rik)
    cj = (rij * inv)[..., None].astype(jnp.bfloat16)
    ck = (rik * inv)[..., None].astype(jnp.bfloat16)
    jj = j_ref[0][..., None].astype(jnp.int16)
    kk = k_ref[0][..., None].astype(jnp.int16)
    aid = lax.broadcasted_iota(jnp.int16, (1, 1, na), 2)
    zero = jnp.zeros((), jnp.bfloat16)
    c_mat = (jnp.where(jj == aid, cj, zero)
             + jnp.where(kk == aid, ck, zero))
    z = jnp.dot(c_mat.reshape(ablk * nbh, na), y_ref[0],
                preferred_element_type=jnp.float32)
    wt = jnp.dot(d_ref[0].reshape(ablk * nbh, g), wft_ref[...],
                 preferred_element_type=jnp.float32)
    wt = wt + bft_ref[0][None, :]
    y2 = jnp.sum((wt * z).reshape(ablk, nbh, f), axis=1)
    o_ref[0] = (jnp.dot(y2, wfo_ref[...],
                        preferred_element_type=jnp.float32)
                + bfo_ref[0][None, :])


_NBUF = 5


def _make_gather(n_rows, f):
    n_chunks = n_rows // _CHUNK
    cpw = n_chunks // _NW
    n_iter = cpw // _NBUF
    mesh = plsc.VectorSubcoreMesh(core_axis_name="c", subcore_axis_name="s")

    @functools.partial(
        pl.kernel,
        mesh=mesh,
        out_type=jax.ShapeDtypeStruct((n_rows, f), jnp.float32),
        scratch_types=[
            pltpu.VMEM((cpw, 1, _CHUNK), jnp.int32),
            pltpu.VMEM((_NBUF, _CHUNK, f), jnp.float32),
        ] + [pltpu.SemaphoreType.DMA] * (2 * _NBUF),
    )
    def gather_kernel(table_hbm, idx_hbm, out_hbm, idx_v, rows_v, *sems):
        gsem = sems[:_NBUF]
        osem = sems[_NBUF:]
        wid = lax.axis_index("s") * _NC + lax.axis_index("c")
        base = wid * cpw
        pltpu.sync_copy(idx_hbm.at[pl.ds(base, cpw)], idx_v)

        def start_gather(bx, c):
            pltpu.async_copy(table_hbm.at[idx_v.at[c, 0]], rows_v.at[bx],
                             gsem[bx])

        def wait_gather(bx, c):
            pltpu.make_async_copy(table_hbm.at[idx_v.at[c, 0]], rows_v.at[bx],
                                  gsem[bx]).wait()

        def start_out(bx, c):
            row0 = (base + c) * _CHUNK
            pltpu.async_copy(rows_v.at[bx], out_hbm.at[pl.ds(row0, _CHUNK)],
                             osem[bx])

        def wait_out(bx):
            pltpu.make_async_copy(rows_v.at[bx],
                                  out_hbm.at[pl.ds(0, _CHUNK)],
                                  osem[bx]).wait()

        for bx in range(_NBUF):
            start_gather(bx, bx)

        def body(g, carry):
            c0 = g * _NBUF
            for bx in range(_NBUF):
                wait_gather(bx, c0 + bx)
                start_out(bx, c0 + bx)
            nxt = c0 + _NBUF

            @pl.when(nxt < cpw)
            def _refill():
                for bx in range(_NBUF):
                    wait_out(bx)
                    start_gather(bx, nxt + bx)

            return carry

        lax.fori_loop(0, n_iter, body, None)
        for bx in range(_NBUF):
            wait_out(bx)

    return gather_kernel


def _combine_body(yj_ref, yk_ref, rij_ref, rik_ref, m_ref, d_ref,
                  wft_ref, bft_ref, wfo_ref, bfo_ref, o_ref):
    _, ablk, nbh, g = d_ref.shape
    f = yj_ref.shape[4]
    rij = rij_ref[0]
    rik = rik_ref[0]
    m = m_ref[0]
    inv = m / (rij + rik)
    cj = (rij * inv)[:, None, :]
    ck = (rik * inv)[:, None, :]
    wt = jnp.dot(d_ref[0].reshape(ablk * nbh, g), wft_ref[...],
                 preferred_element_type=jnp.float32)
    wt = wt.reshape(ablk, nbh, f) + bft_ref[0][None, None, :]
    pj = wt * yj_ref[0, 0]
    pk = wt * yk_ref[0, 0]
    dn = (((2,), (1,)), ((0,), (0,)))
    y2 = (lax.dot_general(cj, pj, dn, preferred_element_type=jnp.float32)
          + lax.dot_general(ck, pk, dn,
                            preferred_element_type=jnp.float32))[:, 0, :]
    o_ref[0] = (jnp.dot(y2, wfo_ref[...],
                        preferred_element_type=jnp.float32)
                + bfo_ref[0][None, :])


def kernel(x, r_double, r_ij, r_ik, r_jk, neighbors, neighbor_mask,
           neighbors_j, neighbors_k, triple_masks, d_ijk, W_in2f, W_f2out,
           b_f2out, W_fd, b_fd, W_ft, b_ft):
    b, na, nbh = neighbors_j.shape
    f = x.shape[2]
    g = d_ijk.shape[3]

    na_sc = 640
    na_tc = na - na_sc
    ablk = 40
    npb = na_sc // ablk
    rows_sc = 2 * b * na_sc * nbh

    y, y_b16, idx_sc = pl.pallas_call(
        _in2f_body,
        out_shape=[jax.ShapeDtypeStruct((b * na, f), jnp.float32),
                   jax.ShapeDtypeStruct((b * na, f), jnp.bfloat16),
                   jax.ShapeDtypeStruct((rows_sc // _CHUNK, 1, _CHUNK),
                                        jnp.int32)],
    )(x.reshape(b * na, f), W_in2f,
      lax.slice_in_dim(neighbors_j, 0, na_sc, axis=1),
      lax.slice_in_dim(neighbors_k, 0, na_sc, axis=1))
    y_b16 = y_b16.reshape(b, na, f)

    outs = []
    yjk = _make_gather(rows_sc, f)(y, idx_sc).reshape(2, b, na_sc, nbh, f)
    out_sc = pl.pallas_call(
        _combine_body,
        grid=(b, npb),
        in_specs=[
            pl.BlockSpec((1, 1, ablk, nbh, f),
                         lambda bi, i: (0, bi, i, 0, 0)),
            pl.BlockSpec((1, 1, ablk, nbh, f),
                         lambda bi, i: (1, bi, i, 0, 0)),
            pl.BlockSpec((1, ablk, nbh), lambda bi, i: (bi, i, 0)),
            pl.BlockSpec((1, ablk, nbh), lambda bi, i: (bi, i, 0)),
            pl.BlockSpec((1, ablk, nbh), lambda bi, i: (bi, i, 0)),
            pl.BlockSpec((1, ablk, nbh, g), lambda bi, i: (bi, i, 0, 0)),
            pl.BlockSpec((g, f), lambda bi, i: (0, 0)),
            pl.BlockSpec((1, f), lambda bi, i: (0, 0)),
            pl.BlockSpec((f, f), lambda bi, i: (0, 0)),
            pl.BlockSpec((1, f), lambda bi, i: (0, 0)),
        ],
        out_specs=pl.BlockSpec((1, ablk, f), lambda bi, i: (bi, i, 0)),
        out_shape=jax.ShapeDtypeStruct((b, na_sc, f), jnp.float32),
        compiler_params=pltpu.CompilerParams(
            dimension_semantics=("parallel", "parallel")),
    )(yjk, yjk, r_ij, r_ik, triple_masks, d_ijk, W_ft,
      b_ft.reshape(1, f), W_f2out, b_f2out.reshape(1, f))
    outs.append(out_sc)

    ablk2 = 8
    nb2 = na_tc // ablk2
    blk0 = na_sc // ablk2
    out_tc = pl.pallas_call(
        _assist_body,
        grid=(b, nb2),
        in_specs=[
            pl.BlockSpec((1, na, f), lambda bi, i: (bi, 0, 0)),
            pl.BlockSpec((1, ablk2, nbh), lambda bi, i: (bi, blk0 + i, 0)),
            pl.BlockSpec((1, ablk2, nbh), lambda bi, i: (bi, blk0 + i, 0)),
            pl.BlockSpec((1, ablk2, nbh), lambda bi, i: (bi, blk0 + i, 0)),
            pl.BlockSpec((1, ablk2, nbh), lambda bi, i: (bi, blk0 + i, 0)),
            pl.BlockSpec((1, ablk2, nbh), lambda bi, i: (bi, blk0 + i, 0)),
            pl.BlockSpec((1, ablk2, nbh, g),
                         lambda bi, i: (bi, blk0 + i, 0, 0)),
            pl.BlockSpec((g, f), lambda bi, i: (0, 0)),
            pl.BlockSpec((1, f), lambda bi, i: (0, 0)),
            pl.BlockSpec((f, f), lambda bi, i: (0, 0)),
            pl.BlockSpec((1, f), lambda bi, i: (0, 0)),
        ],
        out_specs=pl.BlockSpec((1, ablk2, f), lambda bi, i: (bi, i, 0)),
        out_shape=jax.ShapeDtypeStruct((b, na_tc, f), jnp.float32),
        compiler_params=pltpu.CompilerParams(
            dimension_semantics=("parallel", "parallel")),
    )(y_b16, neighbors_j, neighbors_k, r_ij, r_ik, triple_masks, d_ijk,
      W_ft, b_ft.reshape(1, f), W_f2out, b_f2out.reshape(1, f))
    outs.append(out_tc)
    return jnp.concatenate(outs, axis=1)
